# baseline (speedup 1.0000x reference)
import functools

import jax
import jax.numpy as jnp
from jax import lax
from jax.experimental import pallas as pl
from jax.experimental.pallas import tpu as pltpu
from jax.experimental.pallas import tpu_sc as plsc

_K = 1024
_D = 64
_GB = 4
_MT = _GB * 576
_COMMIT = 0.25

_NW = 32
_CHUNK = 96


def _vq_body(z_hbm, w_ref, idx_ref, idxd_ref, loss_ref, zbuf, zsem):
    i = pl.program_id(0)
    ni = pl.num_programs(0)

    def zcopy(step, buf):
        return pltpu.make_async_copy(z_hbm.at[pl.ds(step * _GB, _GB)],
                                     zbuf.at[buf], zsem.at[buf])

    @pl.when(i == 0)
    def _():
        zcopy(0, 0).start()

    @pl.when(i + 1 < ni)
    def _():
        zcopy(i + 1, (i + 1) % 2).start()

    zcopy(i, i % 2).wait()
    zt = zbuf[i % 2].reshape(_MT, _D)
    wt = w_ref[...]
    mm2 = lax.dot_general(zt, wt + wt, (((1,), (1,)), ((), ())),
                          preferred_element_type=jnp.float32)
    z2 = jnp.sum(zt * zt, axis=1, keepdims=True)
    w2 = jnp.sum(wt * wt, axis=1)[None, :]
    dist = (z2 + w2) - mm2

    mind = jnp.min(dist, axis=1, keepdims=True)
    iota = lax.broadcasted_iota(jnp.int32, (1, _K), 1)
    key = lax.bitcast_convert_type(iota + 0x3F800000, jnp.float32)
    minkey = jnp.min(jnp.where(dist == mind, key, 2.0), axis=1)
    idx = lax.bitcast_convert_type(minkey, jnp.int32) - 0x3F800000
    idx_ref[...] = idx[:, None]
    idxd_ref[...] = idx.reshape(1, _MT // 128, 128)

    part = jnp.sum(mind)

    @pl.when(i == 0)
    def _():
        loss_ref[0, 0] = part

    @pl.when(i > 0)
    def _():
        loss_ref[0, 0] += part

    @pl.when(i == ni - 1)
    def _():
        n = _MT * ni * _D
        m = loss_ref[0, 0] / n
        loss_ref[0, 0] = m + _COMMIT * m


def _sc_gather(idx_hbm, w_hbm, out_hbm, w_sh, idx_v, rows_v, gsem):
    b_per_w = idx_hbm.shape[0] // _NW
    nc = b_per_w // _CHUNK
    sid = lax.axis_index("s")
    wid = sid * 2 + lax.axis_index("c")
    base = wid * b_per_w

    @pl.when(sid == 0)
    def _():
        pltpu.sync_copy(w_hbm, w_sh)
    plsc.subcore_barrier()

    pltpu.sync_copy(idx_hbm.at[pl.ds(base, b_per_w)], idx_v)
    copies = [
        pltpu.async_copy(w_sh.at[idx_v.at[pl.ds(c * _CHUNK, _CHUNK)]],
                         rows_v.at[pl.ds(c * _CHUNK, _CHUNK)], gsem)
        for c in range(nc)
    ]
    for cp in copies:
        cp.wait()
    pltpu.sync_copy(rows_v, out_hbm.at[pl.ds(base, b_per_w), pl.ds(0, _D)])


def kernel(z, W):
    b, s, _ = z.shape
    m = b * s
    grid = (b // _GB,)
    idx, idxd, loss = pl.pallas_call(
        _vq_body,
        grid=grid,
        in_specs=[
            pl.BlockSpec(memory_space=pltpu.HBM),
            pl.BlockSpec((_K, _D), lambda i: (0, 0)),
        ],
        scratch_shapes=[
            pltpu.VMEM((2, _GB, s, _D), jnp.float32),
            pltpu.SemaphoreType.DMA((2,)),
        ],
        out_specs=[
            pl.BlockSpec((_MT, 1), lambda i: (i, 0)),
            pl.BlockSpec((1, _MT // 128, 128), lambda i: (i, 0, 0)),
            pl.BlockSpec((1, 1), lambda i: (0, 0), memory_space=pltpu.SMEM),
        ],
        out_shape=[
            jax.ShapeDtypeStruct((m, 1), jnp.int32),
            jax.ShapeDtypeStruct((b // _GB, _MT // 128, 128), jnp.int32),
            jax.ShapeDtypeStruct((1, 1), jnp.float32),
        ],
    )(z, W)

    gather = functools.partial(
        pl.kernel,
        out_type=jax.ShapeDtypeStruct((m, 128), jnp.float32),
        mesh=plsc.VectorSubcoreMesh(core_axis_name="c", subcore_axis_name="s"),
        scratch_types=[
            pltpu.VMEM_SHARED((_K, _D), jnp.float32),
            pltpu.VMEM((m // _NW,), jnp.int32),
            pltpu.VMEM((m // _NW, _D), jnp.float32),
            pltpu.SemaphoreType.DMA,
        ],
        compiler_params=pltpu.CompilerParams(use_tc_tiling_on_sc=False),
    )(_sc_gather)
    zq = gather(idxd.reshape(m), W)[:, :_D]

    return zq.reshape(z.shape), loss[0, 0], idx


# speedup vs baseline: 1.0137x; 1.0137x over previous
import functools

import jax
import jax.numpy as jnp
from jax import lax
from jax.experimental import pallas as pl
from jax.experimental.pallas import tpu as pltpu
from jax.experimental.pallas import tpu_sc as plsc

_K = 1024
_D = 64
_GB = 4
_MT = _GB * 576
_COMMIT = 0.25

_NW = 32
_CHUNK = 96


def _vq_body(z_ref, w_ref, idx_ref, idxd_ref, loss_ref):
    zt = z_ref[...].reshape(_MT, _D)
    wt = w_ref[...]
    mm2 = lax.dot_general(zt, wt + wt, (((1,), (1,)), ((), ())),
                          preferred_element_type=jnp.float32)
    z2 = jnp.sum(zt * zt, axis=1, keepdims=True)
    w2 = jnp.sum(wt * wt, axis=1)[None, :]
    dist = (z2 + w2) - mm2

    mind = jnp.min(dist, axis=1, keepdims=True)
    iota = lax.broadcasted_iota(jnp.int32, (1, _K), 1)
    key = lax.bitcast_convert_type(iota + 0x3F800000, jnp.float32)
    minkey = jnp.min(jnp.where(dist == mind, key, 2.0), axis=1)
    idx = lax.bitcast_convert_type(minkey, jnp.int32) - 0x3F800000
    idx_ref[...] = idx[:, None]
    idxd_ref[...] = idx.reshape(1, _MT // 128, 128)

    part = jnp.sum(mind)
    i = pl.program_id(0)
    ni = pl.num_programs(0)

    @pl.when(i == 0)
    def _():
        loss_ref[0, 0] = part

    @pl.when(i > 0)
    def _():
        loss_ref[0, 0] += part

    @pl.when(i == ni - 1)
    def _():
        n = _MT * ni * _D
        m = loss_ref[0, 0] / n
        loss_ref[0, 0] = m + _COMMIT * m


def _sc_gather(idx_hbm, w_hbm, out_hbm, w_sh, idx_v, rows_v, gsem):
    b_per_w = idx_hbm.shape[0] // _NW
    nc = b_per_w // _CHUNK
    sid = lax.axis_index("s")
    wid = sid * 2 + lax.axis_index("c")
    base = wid * b_per_w

    @pl.when(sid == 0)
    def _():
        pltpu.sync_copy(w_hbm, w_sh)
    plsc.subcore_barrier()

    pltpu.sync_copy(idx_hbm.at[pl.ds(base, b_per_w)], idx_v)
    copies = [
        pltpu.async_copy(w_sh.at[idx_v.at[pl.ds(c * _CHUNK, _CHUNK)]],
                         rows_v.at[pl.ds(c * _CHUNK, _CHUNK)], gsem)
        for c in range(nc)
    ]
    for cp in copies:
        cp.wait()
    pltpu.sync_copy(rows_v, out_hbm.at[pl.ds(base, b_per_w), pl.ds(0, _D)])


def kernel(z, W):
    b, s, _ = z.shape
    m = b * s
    grid = (b // _GB,)
    idx, idxd, loss = pl.pallas_call(
        _vq_body,
        grid=grid,
        in_specs=[
            pl.BlockSpec((_GB, s, _D), lambda i: (i, 0, 0)),
            pl.BlockSpec((_K, _D), lambda i: (0, 0)),
        ],
        out_specs=[
            pl.BlockSpec((_MT, 1), lambda i: (i, 0)),
            pl.BlockSpec((1, _MT // 128, 128), lambda i: (i, 0, 0)),
            pl.BlockSpec((1, 1), lambda i: (0, 0), memory_space=pltpu.SMEM),
        ],
        out_shape=[
            jax.ShapeDtypeStruct((m, 1), jnp.int32),
            jax.ShapeDtypeStruct((b // _GB, _MT // 128, 128), jnp.int32),
            jax.ShapeDtypeStruct((1, 1), jnp.float32),
        ],
    )(z, W)

    gather = functools.partial(
        pl.kernel,
        out_type=jax.ShapeDtypeStruct((m, 128), jnp.float32),
        mesh=plsc.VectorSubcoreMesh(core_axis_name="c", subcore_axis_name="s"),
        scratch_types=[
            pltpu.VMEM_SHARED((_K, _D), jnp.float32),
            pltpu.VMEM((m // _NW,), jnp.int32),
            pltpu.VMEM((m // _NW, _D), jnp.float32),
            pltpu.SemaphoreType.DMA,
        ],
        compiler_params=pltpu.CompilerParams(use_tc_tiling_on_sc=False),
    )(_sc_gather)
    zq = gather(idxd.reshape(m), W)[:, :_D]

    return zq.reshape(z.shape), loss[0, 0], idx
